# initial kernel scaffold (unmeasured)
import jax
import jax.numpy as jnp
from jax import lax
from jax.experimental import pallas as pl
from jax.experimental.pallas import tpu as pltpu

B, S, HD_IN, HD_OUT = 4, 1024, 2048, 4096
S_HALF = S // 2


def kernel(O, Wo):
    O2 = O.reshape(B, S, HD_IN).astype(jnp.bfloat16)
    Wo2 = Wo.astype(jnp.bfloat16)

    def body(o_ref, wo_ref, out_ref, send_buf, recv_buf, send_sems, recv_sems):
        my_x = lax.axis_index("x")
        my_y = lax.axis_index("y")
        partner = (1 - my_x, my_y)

        barrier_sem = pltpu.get_barrier_semaphore()
        pl.semaphore_signal(
            barrier_sem, inc=1,
            device_id=partner, device_id_type=pl.DeviceIdType.MESH,
        )
        pl.semaphore_wait(barrier_sem, 1)

        own_start = my_x * S_HALF
        remote_start = (1 - my_x) * S_HALF

        def rdma(b):
            return pltpu.make_async_remote_copy(
                src_ref=send_buf.at[b],
                dst_ref=recv_buf.at[b],
                send_sem=send_sems.at[b],
                recv_sem=recv_sems.at[b],
                device_id=partner,
                device_id_type=pl.DeviceIdType.MESH,
            )

        for b in range(B):
            o_rem = o_ref[b, pl.ds(remote_start, S_HALF), :]
            p_rem = jnp.dot(
                o_rem, wo_ref[:, :], preferred_element_type=jnp.float32
            )
            send_buf[b] = p_rem.astype(jnp.bfloat16)
            rdma(b).start()

            o_own = o_ref[b, pl.ds(own_start, S_HALF), :]
            out_ref[b] = jnp.dot(
                o_own, wo_ref[:, :], preferred_element_type=jnp.float32
            )

        for b in range(B):
            rdma(b).wait_recv()
            out_ref[b] = out_ref[b] + recv_buf[b].astype(jnp.float32)

        for b in range(B):
            rdma(b).wait_send()

    return pl.pallas_call(
        body,
        out_shape=jax.ShapeDtypeStruct((B, S_HALF, HD_OUT), jnp.float32),
        in_specs=[
            pl.BlockSpec(memory_space=pltpu.VMEM),
            pl.BlockSpec(memory_space=pltpu.VMEM),
        ],
        out_specs=pl.BlockSpec(memory_space=pltpu.VMEM),
        scratch_shapes=[
            pltpu.VMEM((B, S_HALF, HD_OUT), jnp.bfloat16),
            pltpu.VMEM((B, S_HALF, HD_OUT), jnp.bfloat16),
            pltpu.SemaphoreType.DMA((B,)),
            pltpu.SemaphoreType.DMA((B,)),
        ],
        compiler_params=pltpu.CompilerParams(
            collective_id=0,
            vmem_limit_bytes=128 * 1024 * 1024,
        ),
    )(O2, Wo2)


# baseline (device time: 326762 ns/iter reference)
import jax
import jax.numpy as jnp
from jax import lax
from jax.experimental import pallas as pl
from jax.experimental.pallas import tpu as pltpu

B, S, HD_IN, HD_OUT = 4, 1024, 2048, 4096
S_HALF = S // 2


def kernel(O, Wo):
    O2 = O.reshape(B, S, HD_IN).astype(jnp.bfloat16)
    Wo2 = Wo.astype(jnp.bfloat16)

    def body(
        o_hbm, wo_ref, out_hbm,
        o_tile, send_buf, recv_buf, acc,
        send_sems, recv_sems, load_sem, store_sem,
    ):
        my_x = lax.axis_index("x")
        my_y = lax.axis_index("y")
        partner = (1 - my_x, my_y)

        barrier_sem = pltpu.get_barrier_semaphore()
        pl.semaphore_signal(
            barrier_sem, inc=1,
            device_id=partner, device_id_type=pl.DeviceIdType.MESH,
        )
        pl.semaphore_wait(barrier_sem, 1)

        own_start = my_x * S_HALF
        remote_start = (1 - my_x) * S_HALF

        def rdma(b):
            return pltpu.make_async_remote_copy(
                src_ref=send_buf.at[b % 2],
                dst_ref=recv_buf.at[b],
                send_sem=send_sems.at[b % 2],
                recv_sem=recv_sems.at[b],
                device_id=partner,
                device_id_type=pl.DeviceIdType.MESH,
            )

        for b in range(B):
            load = pltpu.make_async_copy(o_hbm.at[b], o_tile, load_sem)
            load.start()
            load.wait()

            if b >= 2:
                rdma(b - 2).wait_send()

            o_rem = o_tile[pl.ds(remote_start, S_HALF), :]
            p_rem = jnp.dot(
                o_rem, wo_ref[:, :], preferred_element_type=jnp.float32
            )
            send_buf[b % 2] = p_rem.astype(jnp.bfloat16)
            rdma(b).start()

            o_own = o_tile[pl.ds(own_start, S_HALF), :]
            acc[...] = jnp.dot(
                o_own, wo_ref[:, :], preferred_element_type=jnp.float32
            )

            rdma(b).wait_recv()
            acc[...] = acc[...] + recv_buf[b].astype(jnp.float32)

            store = pltpu.make_async_copy(acc, out_hbm.at[b], store_sem)
            store.start()
            store.wait()

        for b in range(B - 2, B):
            rdma(b).wait_send()

    return pl.pallas_call(
        body,
        out_shape=jax.ShapeDtypeStruct((B, S_HALF, HD_OUT), jnp.float32),
        in_specs=[
            pl.BlockSpec(memory_space=pl.ANY),
            pl.BlockSpec(memory_space=pltpu.VMEM),
        ],
        out_specs=pl.BlockSpec(memory_space=pl.ANY),
        scratch_shapes=[
            pltpu.VMEM((S, HD_IN), jnp.bfloat16),
            pltpu.VMEM((2, S_HALF, HD_OUT), jnp.bfloat16),
            pltpu.VMEM((B, S_HALF, HD_OUT), jnp.bfloat16),
            pltpu.VMEM((S_HALF, HD_OUT), jnp.float32),
            pltpu.SemaphoreType.DMA((2,)),
            pltpu.SemaphoreType.DMA((B,)),
            pltpu.SemaphoreType.DMA,
            pltpu.SemaphoreType.DMA,
        ],
        compiler_params=pltpu.CompilerParams(
            collective_id=0,
            vmem_limit_bytes=64 * 1024 * 1024,
        ),
    )(O2, Wo2)


# device time: 273525 ns/iter; 1.1946x vs baseline; 1.1946x over previous
import jax
import jax.numpy as jnp
from jax import lax
from jax.experimental import pallas as pl
from jax.experimental.pallas import tpu as pltpu

B, S, HD_IN, HD_OUT = 4, 1024, 2048, 4096
S_HALF = S // 2


def kernel(O, Wo):
    O2 = O.reshape(B, S, HD_IN).astype(jnp.bfloat16)
    Wo2 = Wo.astype(jnp.bfloat16)

    def body(
        o_hbm, wo_ref, out_hbm,
        o_tile, send_buf, recv_buf, own_buf,
        send_sems, recv_sems, load_sem, store_sems, credit_sem,
    ):
        my_x = lax.axis_index("x")
        my_y = lax.axis_index("y")
        partner = (1 - my_x, my_y)

        barrier_sem = pltpu.get_barrier_semaphore()
        pl.semaphore_signal(
            barrier_sem, inc=1,
            device_id=partner, device_id_type=pl.DeviceIdType.MESH,
        )
        pl.semaphore_wait(barrier_sem, 1)

        own_start = my_x * S_HALF
        remote_start = (1 - my_x) * S_HALF

        def rdma(b):
            return pltpu.make_async_remote_copy(
                src_ref=send_buf.at[b % 2],
                dst_ref=recv_buf.at[b % 2],
                send_sem=send_sems.at[b % 2],
                recv_sem=recv_sems.at[b % 2],
                device_id=partner,
                device_id_type=pl.DeviceIdType.MESH,
            )

        def store(b):
            return pltpu.make_async_copy(
                own_buf.at[b % 2], out_hbm.at[b], store_sems.at[b % 2]
            )

        def consume(b):
            rdma(b).wait_recv()
            own_buf[b % 2] = own_buf[b % 2] + recv_buf[b % 2].astype(
                jnp.float32
            )
            pl.semaphore_signal(
                credit_sem, inc=1,
                device_id=partner, device_id_type=pl.DeviceIdType.MESH,
            )
            store(b).start()

        for b in range(B):
            if b >= 2:
                consume(b - 2)

            load = pltpu.make_async_copy(o_hbm.at[b], o_tile, load_sem)
            load.start()
            load.wait()

            p_rem = jnp.dot(
                o_tile[pl.ds(remote_start, S_HALF), :],
                wo_ref[:, :],
                preferred_element_type=jnp.float32,
            )
            if b >= 2:
                rdma(b - 2).wait_send()
            send_buf[b % 2] = p_rem.astype(jnp.bfloat16)
            if b >= 2:
                pl.semaphore_wait(credit_sem, 1)
            rdma(b).start()

            if b >= 2:
                store(b - 2).wait()
            own_buf[b % 2] = jnp.dot(
                o_tile[pl.ds(own_start, S_HALF), :],
                wo_ref[:, :],
                preferred_element_type=jnp.float32,
            )

        for b in range(B - 2, B):
            rdma(b).wait_recv()
            own_buf[b % 2] = own_buf[b % 2] + recv_buf[b % 2].astype(
                jnp.float32
            )
            store(b).start()
        for b in range(B - 2, B):
            rdma(b).wait_send()
            store(b).wait()

    return pl.pallas_call(
        body,
        out_shape=jax.ShapeDtypeStruct((B, S_HALF, HD_OUT), jnp.float32),
        in_specs=[
            pl.BlockSpec(memory_space=pl.ANY),
            pl.BlockSpec(memory_space=pltpu.VMEM),
        ],
        out_specs=pl.BlockSpec(memory_space=pl.ANY),
        scratch_shapes=[
            pltpu.VMEM((S, HD_IN), jnp.bfloat16),
            pltpu.VMEM((2, S_HALF, HD_OUT), jnp.bfloat16),
            pltpu.VMEM((2, S_HALF, HD_OUT), jnp.bfloat16),
            pltpu.VMEM((2, S_HALF, HD_OUT), jnp.float32),
            pltpu.SemaphoreType.DMA((2,)),
            pltpu.SemaphoreType.DMA((2,)),
            pltpu.SemaphoreType.DMA,
            pltpu.SemaphoreType.DMA((2,)),
            pltpu.SemaphoreType.REGULAR,
        ],
        compiler_params=pltpu.CompilerParams(
            collective_id=0,
            vmem_limit_bytes=64 * 1024 * 1024,
        ),
    )(O2, Wo2)


# device time: 226728 ns/iter; 1.4412x vs baseline; 1.2064x over previous
import jax
import jax.numpy as jnp
from jax import lax
from jax.experimental import pallas as pl
from jax.experimental.pallas import tpu as pltpu

B, S, HD_IN, HD_OUT = 4, 1024, 2048, 4096
S_HALF = S // 2
C = HD_OUT // 2


def kernel(O, Wo):
    O2 = O.reshape(B, S, HD_IN).astype(jnp.bfloat16)
    Wo2 = Wo.astype(jnp.bfloat16)

    def body(
        o_hbm, wo_ref, out_hbm,
        o_tile, xsend, xrecv, yrecv, own, stage,
        xsend_sems, xrecv_sems, fsend_sems, yrecv_sems,
        load_sem, store_sem,
    ):
        my_x = lax.axis_index("x")
        my_y = lax.axis_index("y")
        x_nbr = (1 - my_x, my_y)
        y_nbr = (my_x, 1 - my_y)

        barrier_sem = pltpu.get_barrier_semaphore()
        for nbr in (x_nbr, y_nbr):
            pl.semaphore_signal(
                barrier_sem, inc=1,
                device_id=nbr, device_id_type=pl.DeviceIdType.MESH,
            )
        pl.semaphore_wait(barrier_sem, 2)

        own_rows = my_x * S_HALF
        rem_rows = (1 - my_x) * S_HALF

        def xrdma(b):
            return pltpu.make_async_remote_copy(
                src_ref=xsend.at[b % 2],
                dst_ref=xrecv.at[b],
                send_sem=xsend_sems.at[b % 2],
                recv_sem=xrecv_sems.at[b],
                device_id=x_nbr,
                device_id_type=pl.DeviceIdType.MESH,
            )

        def fwd(b):
            return pltpu.make_async_remote_copy(
                src_ref=xrecv.at[b],
                dst_ref=yrecv.at[b],
                send_sem=fsend_sems.at[b],
                recv_sem=yrecv_sems.at[b],
                device_id=y_nbr,
                device_id_type=pl.DeviceIdType.MESH,
            )

        def store(b):
            return pltpu.make_async_copy(stage, out_hbm.at[b], store_sem)

        def consume(b):
            fwd(b).wait_recv()
            if b > 0:
                store(b - 1).wait()

            @pl.when(my_y == 0)
            def _():
                stage[:, :C] = (
                    own[b % 2, :, :C].astype(jnp.float32)
                    + xrecv[b].astype(jnp.float32)
                ).astype(jnp.bfloat16)
                stage[:, C:] = (
                    own[b % 2, :, C:].astype(jnp.float32)
                    + yrecv[b].astype(jnp.float32)
                ).astype(jnp.bfloat16)

            @pl.when(my_y == 1)
            def _():
                stage[:, :C] = (
                    own[b % 2, :, :C].astype(jnp.float32)
                    + yrecv[b].astype(jnp.float32)
                ).astype(jnp.bfloat16)
                stage[:, C:] = (
                    own[b % 2, :, C:].astype(jnp.float32)
                    + xrecv[b].astype(jnp.float32)
                ).astype(jnp.bfloat16)

            store(b).start()

        for b in range(B):
            if b >= 2:
                consume(b - 2)

            load = pltpu.make_async_copy(o_hbm.at[b], o_tile, load_sem)
            load.start()
            load.wait()

            if b >= 2:
                xrdma(b - 2).wait_send()

            o_rem = o_tile[pl.ds(rem_rows, S_HALF), :]

            @pl.when(my_y == 0)
            def _():
                xsend[b % 2] = jnp.dot(
                    o_rem, wo_ref[:, :C],
                    preferred_element_type=jnp.float32,
                ).astype(jnp.bfloat16)

            @pl.when(my_y == 1)
            def _():
                xsend[b % 2] = jnp.dot(
                    o_rem, wo_ref[:, C:],
                    preferred_element_type=jnp.float32,
                ).astype(jnp.bfloat16)

            xrdma(b).start()

            o_own = o_tile[pl.ds(own_rows, S_HALF), :]
            own[b % 2, :, :C] = jnp.dot(
                o_own, wo_ref[:, :C], preferred_element_type=jnp.float32
            ).astype(jnp.bfloat16)
            own[b % 2, :, C:] = jnp.dot(
                o_own, wo_ref[:, C:], preferred_element_type=jnp.float32
            ).astype(jnp.bfloat16)

            xrdma(b).wait_recv()
            fwd(b).start()

        consume(B - 2)
        consume(B - 1)
        for b in range(B - 2, B):
            xrdma(b).wait_send()
        for b in range(B):
            fwd(b).wait_send()
        store(B - 1).wait()

    return pl.pallas_call(
        body,
        out_shape=jax.ShapeDtypeStruct((B, S_HALF, HD_OUT), jnp.bfloat16),
        in_specs=[
            pl.BlockSpec(memory_space=pl.ANY),
            pl.BlockSpec(memory_space=pltpu.VMEM),
        ],
        out_specs=pl.BlockSpec(memory_space=pl.ANY),
        scratch_shapes=[
            pltpu.VMEM((S, HD_IN), jnp.bfloat16),
            pltpu.VMEM((2, S_HALF, C), jnp.bfloat16),
            pltpu.VMEM((B, S_HALF, C), jnp.bfloat16),
            pltpu.VMEM((B, S_HALF, C), jnp.bfloat16),
            pltpu.VMEM((2, S_HALF, HD_OUT), jnp.bfloat16),
            pltpu.VMEM((S_HALF, HD_OUT), jnp.bfloat16),
            pltpu.SemaphoreType.DMA((2,)),
            pltpu.SemaphoreType.DMA((B,)),
            pltpu.SemaphoreType.DMA((B,)),
            pltpu.SemaphoreType.DMA((B,)),
            pltpu.SemaphoreType.DMA,
            pltpu.SemaphoreType.DMA,
        ],
        compiler_params=pltpu.CompilerParams(
            collective_id=0,
            vmem_limit_bytes=64 * 1024 * 1024,
        ),
    )(O2, Wo2)
